# baseline (speedup 1.0000x reference)
import functools

import jax
import jax.numpy as jnp
from jax import lax
from jax.experimental import pallas as pl
from jax.experimental.pallas import tpu as pltpu
from jax.experimental.pallas import tpu_sc as plsc

N = 100000
E = 3200000
C = 8
THR = 0.5
EPS = 1e-07

R = 1024
NBLK = 98
NPAD = R * NBLK
PADROW = N

NW = 32
EPW = E // NW
CH = 2000
NCH = EPW // CH
GRP = CH // 16
CAP = CH + 224
DB = 128


def _prep_body(p_ref, cl_ref, tg_ref, u_ref, v_ref, pk_ref):
    i = pl.program_id(0)
    p = p_ref[...]
    cl = cl_ref[...]
    tg = tg_ref[...]
    col = i * R + lax.broadcasted_iota(jnp.int32, (1, R), 1)
    valid = col < N
    validc = jnp.broadcast_to(valid, (C, R))
    L = jnp.log(cl + EPS)
    tsafe = jnp.where(tg > 0, tg, 1.0)
    s = jnp.sum(jnp.where(tg > 0, tg * jnp.log(tsafe), 0.0), axis=0,
                keepdims=True)
    m = jnp.max(tg, axis=0, keepdims=True)
    rows = lax.broadcasted_iota(jnp.int32, (C, R), 0)
    amax = jnp.min(jnp.where(tg == m, rows, C), axis=0, keepdims=True)
    code = amax + C * (p >= THR).astype(jnp.int32)
    sbits = lax.bitcast_convert_type(s, jnp.int32)
    packed = jnp.where(valid, (sbits & jnp.int32(-16)) | code, 0)
    Tm = jnp.where(validc, tg, 0.0)
    Lm = jnp.where(validc, L, 0.0)
    u_ref[0:C, :] = Tm
    u_ref[C:2 * C, :] = Lm
    v_ref[0:C, :] = Lm
    v_ref[C:2 * C, :] = Tm
    for k in range(8):
        pk_ref[k:k + 1, :] = packed[:, 128 * k:128 * (k + 1)]


_prep = pl.pallas_call(
    _prep_body,
    grid=(NBLK,),
    in_specs=[
        pl.BlockSpec((1, R), lambda i: (0, i)),
        pl.BlockSpec((C, R), lambda i: (0, i)),
        pl.BlockSpec((C, R), lambda i: (0, i)),
    ],
    out_specs=[
        pl.BlockSpec((2 * C, R), lambda i: (0, i)),
        pl.BlockSpec((2 * C, R), lambda i: (0, i)),
        pl.BlockSpec((8, 128), lambda i: (i, 0)),
    ],
    out_shape=[
        jax.ShapeDtypeStruct((2 * C, NPAD), jnp.float32),
        jax.ShapeDtypeStruct((2 * C, NPAD), jnp.float32),
        jax.ShapeDtypeStruct((NPAD // 128, 128), jnp.int32),
    ],
)


SLAB2 = (NPAD // 8) // NW
SLAB = NPAD // NW


def _retile_body(ut, vt, utab, vtab, bufa, bufb):
    wid = lax.axis_index("s") * 2 + lax.axis_index("c")
    lane = lax.iota(jnp.int32, 16)

    def one(src, dst):
        for r in range(2 * C):
            pltpu.sync_copy(src.at[r, pl.ds(wid * SLAB, SLAB)],
                            bufa.at[pl.ds(r * SLAB, SLAB)])

        def tr(j, dummy):
            bufb[j, :] = plsc.load_gather(bufa, [j + SLAB * lane])
            return dummy
        lax.fori_loop(0, SLAB, tr, 0)
        pltpu.sync_copy(bufb, dst.at[pl.ds(wid * SLAB, SLAB), :])

    one(ut, utab)
    one(vt, vtab)


@functools.cache
def _retile():
    return functools.partial(
        pl.kernel,
        mesh=plsc.VectorSubcoreMesh(core_axis_name="c", subcore_axis_name="s"),
        compiler_params=pltpu.CompilerParams(
            needs_layout_passes=False, use_tc_tiling_on_sc=False),
        out_type=[jax.ShapeDtypeStruct((NPAD, 2 * C), jnp.float32),
                  jax.ShapeDtypeStruct((NPAD, 2 * C), jnp.float32)],
        scratch_types=[
            pltpu.VMEM((SLAB * 2 * C,), jnp.float32),
            pltpu.VMEM((SLAB, 2 * C), jnp.float32),
        ],
    )(_retile_body)


def _edge_body(ed0_hbm, ed1_hbm, packed_hbm, utab_hbm, vtab_hbm, out_hbm,
               pk_v, ed0_v, ed1_v, uidx, vidx, urows, vrows, part_v,
               sem1, sem2, sem3, sem4):
    wid = lax.axis_index("s") * 2 + lax.axis_index("c")
    base = wid * EPW
    pltpu.sync_copy(packed_hbm, pk_v)
    padv = jnp.full((16,), PADROW, jnp.int32)

    def fire_edges(c):
        p = c & 1
        pltpu.make_async_copy(ed0_hbm.at[pl.ds(base + c * CH, CH)],
                              ed0_v.at[p], sem3.at[p]).start()
        pltpu.make_async_copy(ed1_hbm.at[pl.ds(base + c * CH, CH)],
                              ed1_v.at[p], sem4.at[p]).start()

    fire_edges(0)

    def chunk_body(c, carry):
        np_acc, s_acc, cr_acc = carry

        @pl.when(c + 1 < NCH)
        def _():
            fire_edges(c + 1)
        ep = c & 1
        pltpu.make_async_copy(ed0_hbm.at[pl.ds(base + c * CH, CH)],
                              ed0_v.at[ep], sem3.at[ep]).wait()
        pltpu.make_async_copy(ed1_hbm.at[pl.ds(base + c * CH, CH)],
                              ed1_v.at[ep], sem4.at[ep]).wait()

        def grp_body(g, car):
            cur, npa, sa = car
            e0 = ed0_v[ep, pl.ds(g * 16, 16)]
            e1 = ed1_v[ep, pl.ds(g * 16, 16)]
            w0 = plsc.load_gather(pk_v, [lax.shift_right_logical(e0, 7),
                                         e0 & 127])
            w1 = plsc.load_gather(pk_v, [lax.shift_right_logical(e1, 7),
                                         e1 & 127])
            msk = ((w0 ^ w1) & 15) == 0
            npa = npa + msk.astype(jnp.int32)
            s0 = plsc.bitcast(w0 & jnp.int32(-16), jnp.float32)
            s1 = plsc.bitcast(w1 & jnp.int32(-16), jnp.float32)
            sa = sa + jnp.where(msk, s0 + s1, 0.0)
            plsc.store_compressed(uidx.at[pl.ds(cur, 16)], e1, mask=msk)
            plsc.store_compressed(vidx.at[pl.ds(cur, 16)], e0, mask=msk)
            cnt = jnp.sum(msk.astype(jnp.int32))
            return (cur + cnt, npa, sa)

        cur, np_acc, s_acc = lax.fori_loop(
            0, GRP, grp_body, (jnp.int32(0), np_acc, s_acc))

        for k in range(DB // 16):
            uidx[pl.ds(cur + k * 16, 16)] = padv
            vidx[pl.ds(cur + k * 16, 16)] = padv

        nb = lax.shift_right_logical(cur + (DB - 1), 7)

        def fire(b):
            p = b & 1
            pltpu.make_async_copy(
                utab_hbm.at[uidx.at[pl.ds(b * DB, DB)]], urows.at[p],
                sem1.at[p]).start()
            pltpu.make_async_copy(
                vtab_hbm.at[vidx.at[pl.ds(b * DB, DB)]], vrows.at[p],
                sem2.at[p]).start()

        @pl.when(nb > 0)
        def _():
            fire(0)

        def dr_body(b, cra):
            @pl.when(b + 1 < nb)
            def _():
                fire(b + 1)
            p = b & 1
            pltpu.make_async_copy(
                utab_hbm.at[uidx.at[pl.ds(b * DB, DB)]], urows.at[p],
                sem1.at[p]).wait()
            pltpu.make_async_copy(
                vtab_hbm.at[vidx.at[pl.ds(b * DB, DB)]], vrows.at[p],
                sem2.at[p]).wait()

            def rowb(j, a):
                return a + urows[p, j, :] * vrows[p, j, :]
            return lax.fori_loop(0, DB, rowb, cra, unroll=4)

        cr_acc = lax.fori_loop(0, nb, dr_body, cr_acc)
        return (np_acc, s_acc, cr_acc)

    np_acc, s_acc, cr_acc = lax.fori_loop(
        0, NCH, chunk_body,
        (jnp.zeros((16,), jnp.int32), jnp.zeros((16,), jnp.float32),
         jnp.zeros((16,), jnp.float32)))
    part_v[0, :] = np_acc.astype(jnp.float32)
    part_v[1, :] = s_acc
    part_v[2, :] = cr_acc
    pltpu.sync_copy(part_v, out_hbm.at[wid])


@functools.cache
def _edge():
    return functools.partial(
        pl.kernel,
        mesh=plsc.VectorSubcoreMesh(core_axis_name="c", subcore_axis_name="s"),
        compiler_params=pltpu.CompilerParams(
            needs_layout_passes=False, use_tc_tiling_on_sc=False),
        out_type=jax.ShapeDtypeStruct((NW, 3, 16), jnp.float32),
        scratch_types=[
            pltpu.VMEM((NPAD // 128, 128), jnp.int32),
            pltpu.VMEM((2, CH), jnp.int32),
            pltpu.VMEM((2, CH), jnp.int32),
            pltpu.VMEM((CAP,), jnp.int32),
            pltpu.VMEM((CAP,), jnp.int32),
            pltpu.VMEM((2, DB, 2 * C), jnp.float32),
            pltpu.VMEM((2, DB, 2 * C), jnp.float32),
            pltpu.VMEM((3, 16), jnp.float32),
            pltpu.SemaphoreType.DMA((2,)),
            pltpu.SemaphoreType.DMA((2,)),
            pltpu.SemaphoreType.DMA((2,)),
            pltpu.SemaphoreType.DMA((2,)),
        ],
    )(_edge_body)


def kernel(edges_nn, probas, clusters, targets):
    edges = edges_nn.astype(jnp.int32)
    e0 = edges[:, 0]
    e1 = edges[:, 1]
    ut, vt, packed = _prep(probas.reshape(1, N), clusters.T, targets.T)
    u_tab, v_tab = _retile()(ut, vt)
    parts = _edge()(e0, e1, packed, u_tab, v_tab)
    n_pos = jnp.sum(parts[:, 0])
    s_sum = jnp.sum(parts[:, 1])
    cross = jnp.sum(parts[:, 2])
    loss = (s_sum - cross) / (2.0 * C * n_pos * n_pos)
    return jnp.where(n_pos > 0, loss, jnp.float32(0.0))


# speedup vs baseline: 1.6629x; 1.6629x over previous
import functools

import jax
import jax.numpy as jnp
from jax import lax
from jax.experimental import pallas as pl
from jax.experimental.pallas import tpu as pltpu
from jax.experimental.pallas import tpu_sc as plsc

N = 100000
E = 3200000
C = 8
THR = 0.5
EPS = 1e-07

R = 1024
NBLK = 98
NPAD = R * NBLK
PADROW = N

NW = 32
EPW = E // NW
CH = 4000
NCH = EPW // CH
GRP = CH // 16
CAP = CH + 224
DB = 64
DBLOG = 6


def _prep_body(p_ref, cl_ref, tg_ref, u_ref, v_ref, pk_ref):
    i = pl.program_id(0)
    p = p_ref[...]
    cl = cl_ref[...]
    tg = tg_ref[...]
    col = i * R + lax.broadcasted_iota(jnp.int32, (1, R), 1)
    valid = col < N
    validc = jnp.broadcast_to(valid, (C, R))
    L = jnp.log(cl + EPS)
    tsafe = jnp.where(tg > 0, tg, 1.0)
    s = jnp.sum(jnp.where(tg > 0, tg * jnp.log(tsafe), 0.0), axis=0,
                keepdims=True)
    m = jnp.max(tg, axis=0, keepdims=True)
    rows = lax.broadcasted_iota(jnp.int32, (C, R), 0)
    amax = jnp.min(jnp.where(tg == m, rows, C), axis=0, keepdims=True)
    code = amax + C * (p >= THR).astype(jnp.int32)
    sbits = lax.bitcast_convert_type(s, jnp.int32)
    packed = jnp.where(valid, (sbits & jnp.int32(-16)) | code, 0)
    Tm = jnp.where(validc, tg, 0.0)
    Lm = jnp.where(validc, L, 0.0)
    u_ref[0:C, :] = Tm
    u_ref[C:2 * C, :] = Lm
    v_ref[0:C, :] = Lm
    v_ref[C:2 * C, :] = Tm
    for k in range(8):
        pk_ref[k:k + 1, :] = packed[:, 128 * k:128 * (k + 1)]


_prep = pl.pallas_call(
    _prep_body,
    grid=(NBLK,),
    in_specs=[
        pl.BlockSpec((1, R), lambda i: (0, i)),
        pl.BlockSpec((C, R), lambda i: (0, i)),
        pl.BlockSpec((C, R), lambda i: (0, i)),
    ],
    out_specs=[
        pl.BlockSpec((2 * C, R), lambda i: (0, i)),
        pl.BlockSpec((2 * C, R), lambda i: (0, i)),
        pl.BlockSpec((8, 128), lambda i: (i, 0)),
    ],
    out_shape=[
        jax.ShapeDtypeStruct((2 * C, NPAD), jnp.float32),
        jax.ShapeDtypeStruct((2 * C, NPAD), jnp.float32),
        jax.ShapeDtypeStruct((NPAD // 128, 128), jnp.int32),
    ],
)


SLAB2 = (NPAD // 8) // NW
SLAB = NPAD // NW


def _retile_body(ut, vt, utab, vtab, bufa, bufb):
    wid = lax.axis_index("s") * 2 + lax.axis_index("c")
    lane = lax.iota(jnp.int32, 16)

    def one(src, dst):
        for r in range(2 * C):
            pltpu.sync_copy(src.at[r, pl.ds(wid * SLAB, SLAB)],
                            bufa.at[pl.ds(r * SLAB, SLAB)])

        def tr(j, dummy):
            bufb[j, :] = plsc.load_gather(bufa, [j + SLAB * lane])
            return dummy
        lax.fori_loop(0, SLAB, tr, 0)
        pltpu.sync_copy(bufb, dst.at[pl.ds(wid * SLAB, SLAB), :])

    one(ut, utab)
    one(vt, vtab)


@functools.cache
def _retile():
    return functools.partial(
        pl.kernel,
        mesh=plsc.VectorSubcoreMesh(core_axis_name="c", subcore_axis_name="s"),
        compiler_params=pltpu.CompilerParams(
            needs_layout_passes=False, use_tc_tiling_on_sc=False),
        out_type=[jax.ShapeDtypeStruct((NPAD, 2 * C), jnp.float32),
                  jax.ShapeDtypeStruct((NPAD, 2 * C), jnp.float32)],
        scratch_types=[
            pltpu.VMEM((SLAB * 2 * C,), jnp.float32),
            pltpu.VMEM((SLAB, 2 * C), jnp.float32),
        ],
    )(_retile_body)


def _edge_body(ed0_hbm, ed1_hbm, packed_hbm, utab_hbm, vtab_hbm, out_hbm,
               pk_v, ed0_v, ed1_v, uidx, vidx, urows, vrows, part_v,
               sem1, sem2, sem3, sem4):
    wid = lax.axis_index("s") * 2 + lax.axis_index("c")
    base = wid * EPW
    pltpu.sync_copy(packed_hbm, pk_v)
    padv = jnp.full((16,), PADROW, jnp.int32)

    def fire_edges(c):
        p = c & 1
        pltpu.make_async_copy(ed0_hbm.at[pl.ds(base + c * CH, CH)],
                              ed0_v.at[p], sem3.at[p]).start()
        pltpu.make_async_copy(ed1_hbm.at[pl.ds(base + c * CH, CH)],
                              ed1_v.at[p], sem4.at[p]).start()

    fire_edges(0)

    def chunk_body(c, carry):
        np_acc, s_acc, cr_acc = carry

        @pl.when(c + 1 < NCH)
        def _():
            fire_edges(c + 1)
        ep = c & 1
        pltpu.make_async_copy(ed0_hbm.at[pl.ds(base + c * CH, CH)],
                              ed0_v.at[ep], sem3.at[ep]).wait()
        pltpu.make_async_copy(ed1_hbm.at[pl.ds(base + c * CH, CH)],
                              ed1_v.at[ep], sem4.at[ep]).wait()

        def grp_body(g, car):
            cur, npa, sa = car
            e0 = ed0_v[ep, pl.ds(g * 16, 16)]
            e1 = ed1_v[ep, pl.ds(g * 16, 16)]
            w0 = plsc.load_gather(pk_v, [lax.shift_right_logical(e0, 7),
                                         e0 & 127])
            w1 = plsc.load_gather(pk_v, [lax.shift_right_logical(e1, 7),
                                         e1 & 127])
            msk = ((w0 ^ w1) & 15) == 0
            npa = npa + msk.astype(jnp.int32)
            s0 = plsc.bitcast(w0 & jnp.int32(-16), jnp.float32)
            s1 = plsc.bitcast(w1 & jnp.int32(-16), jnp.float32)
            sa = sa + jnp.where(msk, s0 + s1, 0.0)
            plsc.store_compressed(uidx.at[pl.ds(cur, 16)], e1, mask=msk)
            plsc.store_compressed(vidx.at[pl.ds(cur, 16)], e0, mask=msk)
            cnt = jnp.sum(msk.astype(jnp.int32))
            return (cur + cnt, npa, sa)

        cur, np_acc, s_acc = lax.fori_loop(
            0, GRP, grp_body, (jnp.int32(0), np_acc, s_acc))

        for k in range(DB // 16):
            uidx[pl.ds(cur + k * 16, 16)] = padv
            vidx[pl.ds(cur + k * 16, 16)] = padv

        nb = lax.shift_right_logical(cur + (DB - 1), DBLOG)

        def fire(b):
            p = b & 1
            pltpu.make_async_copy(
                utab_hbm.at[uidx.at[pl.ds(b * DB, DB)]], urows.at[p],
                sem1.at[p]).start()
            pltpu.make_async_copy(
                vtab_hbm.at[vidx.at[pl.ds(b * DB, DB)]], vrows.at[p],
                sem2.at[p]).start()

        @pl.when(nb > 0)
        def _():
            fire(0)

        def dr_body(b, cra):
            @pl.when(b + 1 < nb)
            def _():
                fire(b + 1)
            p = b & 1
            pltpu.make_async_copy(
                utab_hbm.at[uidx.at[pl.ds(b * DB, DB)]], urows.at[p],
                sem1.at[p]).wait()
            pltpu.make_async_copy(
                vtab_hbm.at[vidx.at[pl.ds(b * DB, DB)]], vrows.at[p],
                sem2.at[p]).wait()

            def rowb(j, a):
                return a + urows[p, j, :] * vrows[p, j, :]
            return lax.fori_loop(0, DB, rowb, cra, unroll=4)

        cr_acc = lax.fori_loop(0, nb, dr_body, cr_acc)
        return (np_acc, s_acc, cr_acc)

    np_acc, s_acc, cr_acc = lax.fori_loop(
        0, NCH, chunk_body,
        (jnp.zeros((16,), jnp.int32), jnp.zeros((16,), jnp.float32),
         jnp.zeros((16,), jnp.float32)))
    part_v[0, :] = np_acc.astype(jnp.float32)
    part_v[1, :] = s_acc
    part_v[2, :] = cr_acc
    pltpu.sync_copy(part_v, out_hbm.at[wid])


@functools.cache
def _edge():
    return functools.partial(
        pl.kernel,
        mesh=plsc.VectorSubcoreMesh(core_axis_name="c", subcore_axis_name="s"),
        compiler_params=pltpu.CompilerParams(
            needs_layout_passes=False, use_tc_tiling_on_sc=False),
        out_type=jax.ShapeDtypeStruct((NW, 3, 16), jnp.float32),
        scratch_types=[
            pltpu.VMEM((NPAD // 128, 128), jnp.int32),
            pltpu.VMEM((2, CH), jnp.int32),
            pltpu.VMEM((2, CH), jnp.int32),
            pltpu.VMEM((CAP,), jnp.int32),
            pltpu.VMEM((CAP,), jnp.int32),
            pltpu.VMEM((2, DB, 2 * C), jnp.float32),
            pltpu.VMEM((2, DB, 2 * C), jnp.float32),
            pltpu.VMEM((3, 16), jnp.float32),
            pltpu.SemaphoreType.DMA((2,)),
            pltpu.SemaphoreType.DMA((2,)),
            pltpu.SemaphoreType.DMA((2,)),
            pltpu.SemaphoreType.DMA((2,)),
        ],
    )(_edge_body)


def kernel(edges_nn, probas, clusters, targets):
    edges = edges_nn.astype(jnp.int32)
    e0 = edges[:, 0]
    e1 = edges[:, 1]
    ut, vt, packed = _prep(probas.reshape(1, N), clusters.T, targets.T)
    u_tab, v_tab = _retile()(ut, vt)
    parts = _edge()(e0, e1, packed, u_tab, v_tab)
    n_pos = jnp.sum(parts[:, 0])
    s_sum = jnp.sum(parts[:, 1])
    cross = jnp.sum(parts[:, 2])
    loss = (s_sum - cross) / (2.0 * C * n_pos * n_pos)
    return jnp.where(n_pos > 0, loss, jnp.float32(0.0))
